# baseline (speedup 1.0000x reference)
import jax
import jax.numpy as jnp
from jax import lax
from jax.experimental import pallas as pl
from jax.experimental.pallas import tpu as pltpu
from jax.experimental.pallas import tpu_sc as plsc

D_MODEL = 128
_L = 16


def _pe_lookup_body(beta_hbm, pe_hbm, dims_hbm, out_hbm, beta_v, dims_v, rows_v, sem):
    cid = lax.axis_index("c")
    sid = lax.axis_index("s")

    @pl.when(jnp.logical_and(cid == 0, sid == 0))
    def _():
        pltpu.sync_copy(beta_hbm, beta_v)
        pltpu.sync_copy(dims_hbm, dims_v)
        prod = beta_v[...] * dims_v[...]
        idx = prod[0] + prod[1] + prod[2]
        idx_vec = jnp.broadcast_to(idx, (_L,))
        pltpu.async_copy(pe_hbm.at[idx_vec], rows_v, sem).wait()
        pltpu.sync_copy(rows_v.at[0], out_hbm)


def kernel(beta, pe, beta_dims):
    max_len = pe.shape[0]
    table = pe.reshape(max_len, D_MODEL)
    beta16 = jnp.zeros((_L,), jnp.int32).at[:3].set(beta)
    dims16 = jnp.zeros((_L,), jnp.int32).at[:3].set(beta_dims)

    mesh = plsc.VectorSubcoreMesh(core_axis_name="c", subcore_axis_name="s")
    out = pl.kernel(
        _pe_lookup_body,
        out_type=jax.ShapeDtypeStruct((D_MODEL,), jnp.float32),
        mesh=mesh,
        scratch_types=[
            pltpu.VMEM((_L,), jnp.int32),
            pltpu.VMEM((_L,), jnp.int32),
            pltpu.VMEM((_L, D_MODEL), jnp.float32),
            pltpu.SemaphoreType.DMA,
        ],
    )(beta16, table, dims16)
    return out.reshape(1, D_MODEL)


# speedup vs baseline: 1.0615x; 1.0615x over previous
import jax
import jax.numpy as jnp
from jax import lax
from jax.experimental import pallas as pl
from jax.experimental.pallas import tpu as pltpu
from jax.experimental.pallas import tpu_sc as plsc

D_MODEL = 128
_L = 16


def _pe_lookup_body(beta_hbm, pe_hbm, dims_hbm, out_hbm, buf, sem):
    cid = lax.axis_index("c")
    sid = lax.axis_index("s")

    @pl.when(jnp.logical_and(cid == 0, sid == 0))
    def _():
        cp1 = pltpu.async_copy(beta_hbm, buf.at[pl.ds(0, 3)], sem)
        cp2 = pltpu.async_copy(dims_hbm, buf.at[pl.ds(8, 3)], sem)
        cp1.wait()
        cp2.wait()
        v = buf[...]
        idx = v[0] * v[8] + v[1] * v[9] + v[2] * v[10]
        pltpu.sync_copy(pe_hbm.at[pl.ds(idx, 1)], out_hbm)


def kernel(beta, pe, beta_dims):
    max_len = pe.shape[0]
    table = pe.reshape(max_len, D_MODEL)

    mesh = plsc.VectorSubcoreMesh(core_axis_name="c", subcore_axis_name="s")
    out = pl.kernel(
        _pe_lookup_body,
        out_type=jax.ShapeDtypeStruct((1, D_MODEL), jnp.float32),
        mesh=mesh,
        scratch_types=[
            pltpu.VMEM((_L,), jnp.int32),
            pltpu.SemaphoreType.DMA,
        ],
    )(beta, table, beta_dims)
    return out


# speedup vs baseline: 1.1802x; 1.1118x over previous
import jax
import jax.numpy as jnp
from jax import lax
from jax.experimental import pallas as pl
from jax.experimental.pallas import tpu as pltpu
from jax.experimental.pallas import tpu_sc as plsc

D_MODEL = 128
_L = 16


def _pe_lookup_body(beta_hbm, pe_hbm, dims_hbm, out_hbm, buf, sem):
    cp1 = pltpu.async_copy(beta_hbm, buf.at[pl.ds(0, 3)], sem)
    cp2 = pltpu.async_copy(dims_hbm, buf.at[pl.ds(8, 3)], sem)
    cp1.wait()
    cp2.wait()
    v = buf[...]
    idx = v[0] * v[8] + v[1] * v[9] + v[2] * v[10]
    pltpu.sync_copy(pe_hbm.at[pl.ds(idx, 1)], out_hbm)


def kernel(beta, pe, beta_dims):
    max_len = pe.shape[0]
    table = pe.reshape(max_len, D_MODEL)

    mesh = plsc.VectorSubcoreMesh(
        core_axis_name="c", subcore_axis_name="s", num_cores=1, num_subcores=1
    )
    out = pl.kernel(
        _pe_lookup_body,
        out_type=jax.ShapeDtypeStruct((1, D_MODEL), jnp.float32),
        mesh=mesh,
        scratch_types=[
            pltpu.VMEM((_L,), jnp.int32),
            pltpu.SemaphoreType.DMA,
        ],
    )(beta, table, beta_dims)
    return out


# speedup vs baseline: 1.2908x; 1.0938x over previous
import jax
import jax.numpy as jnp
from jax import lax
from jax.experimental import pallas as pl
from jax.experimental.pallas import tpu as pltpu
from jax.experimental.pallas import tpu_sc as plsc

D_MODEL = 128


def _pe_lookup_body(beta_hbm, pe_hbm, dims_hbm, out_hbm, beta_s, dims_s, sem):
    cp1 = pltpu.async_copy(beta_hbm, beta_s, sem)
    cp2 = pltpu.async_copy(dims_hbm, dims_s, sem)
    cp1.wait()
    cp2.wait()
    idx = (
        beta_s[0] * dims_s[0]
        + beta_s[1] * dims_s[1]
        + beta_s[2] * dims_s[2]
    )
    pltpu.sync_copy(pe_hbm.at[pl.ds(idx, 1)], out_hbm)


def kernel(beta, pe, beta_dims):
    max_len = pe.shape[0]
    table = pe.reshape(max_len, D_MODEL)

    mesh = plsc.ScalarSubcoreMesh(axis_name="c", num_cores=1)
    out = pl.kernel(
        _pe_lookup_body,
        out_type=jax.ShapeDtypeStruct((1, D_MODEL), jnp.float32),
        mesh=mesh,
        scratch_types=[
            pltpu.SMEM((3,), jnp.int32),
            pltpu.SMEM((3,), jnp.int32),
            pltpu.SemaphoreType.DMA,
        ],
    )(beta, table, beta_dims)
    return out
